# baseline (speedup 1.0000x reference)
import functools

import jax
import jax.numpy as jnp
from jax import lax
from jax.experimental import pallas as pl
from jax.experimental.pallas import tpu as pltpu
from jax.experimental.pallas import tpu_sc as plsc

N = 10000
NPAD = 10240
E = 640000
DIN = 128
DH = 256
DOUT = 128
EPS = 1e-5

NC, NS = 2, 16
NT = NC * NS
EPT = E // NT
CH = 80
DHALF = DIN // NC
EPS_T = E // NS
NCHUNK = EPS_T // CH
NBUF = 2
RPT = N // NS

_MESH = plsc.VectorSubcoreMesh(
    core_axis_name="c", subcore_axis_name="s", num_cores=NC, num_subcores=NS
)
_SC_PARAMS = pltpu.CompilerParams(
    needs_layout_passes=False, use_tc_tiling_on_sc=False
)


@functools.partial(
    pl.kernel,
    out_type=jax.ShapeDtypeStruct((NT, NPAD), jnp.float32),
    mesh=_MESH,
    compiler_params=_SC_PARAMS,
    scratch_types=[
        pltpu.VMEM((EPT,), jnp.int32),
        pltpu.VMEM((NPAD,), jnp.float32),
    ],
)
def _deg_kernel(dst_hbm, zpad_hbm, out_hbm, dst_v, cnt_v):
    c = lax.axis_index("c")
    s = lax.axis_index("s")
    gid = c * NS + s
    pltpu.sync_copy(zpad_hbm, cnt_v)
    pltpu.sync_copy(dst_hbm.at[pl.ds(gid * EPT, EPT)], dst_v)
    ones = jnp.ones((16,), jnp.float32)

    def count_body(i, carry):
        idx = dst_v[pl.ds(i * 16, 16)]
        plsc.addupdate_scatter(cnt_v, [idx], ones)
        return carry

    lax.fori_loop(0, EPT // 16, count_body, 0)
    pltpu.sync_copy(cnt_v, out_hbm.at[gid])


@functools.partial(
    pl.kernel,
    out_type=jax.ShapeDtypeStruct((NC, N, DHALF), jnp.float32),
    mesh=_MESH,
    compiler_params=_SC_PARAMS,
    scratch_types=[
        pltpu.VMEM((NCHUNK, CH), jnp.int32),
        pltpu.VMEM((NCHUNK, CH), jnp.int32),
        [pltpu.VMEM((CH, DHALF), jnp.float32)] * NBUF,
        [pltpu.SemaphoreType.DMA] * NBUF,
        pltpu.SemaphoreType.DMA,
        pltpu.VMEM_SHARED((N, DHALF), jnp.float32),
    ],
)
def _agg_kernel(table0_hbm, table1_hbm, src_hbm, dst_hbm, z2d_hbm, out_hbm,
                srcb, dstb, rows, gsem, psem, acc):
    c = lax.axis_index("c")
    s = lax.axis_index("s")
    zcopy = pltpu.async_copy(z2d_hbm.at[pl.ds(s * RPT, RPT)],
                             acc.at[pl.ds(s * RPT, RPT)], psem)
    scopy = pltpu.async_copy(src_hbm.at[s], srcb, gsem[0])
    dcopy = pltpu.async_copy(dst_hbm.at[s], dstb, gsem[1])
    scopy.wait()
    dcopy.wait()
    zcopy.wait()

    def run(table_hbm):
        pltpu.async_copy(table_hbm.at[srcb.at[0]], rows[0], gsem[0])
        pltpu.async_copy(table_hbm.at[srcb.at[1]], rows[1], gsem[1])
        plsc.subcore_barrier()

        def step(kk, b):
            pltpu.make_async_copy(table_hbm.at[srcb.at[kk]], rows[b],
                                  gsem[b]).wait()
            pltpu.sync_copy(rows[b], acc.at[dstb.at[kk]], add=True)

            @pl.when(kk + 2 < NCHUNK)
            def _():
                pltpu.async_copy(table_hbm.at[srcb.at[kk + 2]], rows[b],
                                 gsem[b])

        def pair(k, carry):
            step(2 * k, 0)
            step(2 * k + 1, 1)
            return carry

        lax.fori_loop(0, NCHUNK // NBUF, pair, 0)
        plsc.subcore_barrier()
        pltpu.sync_copy(acc.at[pl.ds(s * RPT, RPT)],
                        out_hbm.at[c, pl.ds(s * RPT, RPT)])

    @pl.when(c == 0)
    def _():
        run(table0_hbm)

    @pl.when(c == 1)
    def _():
        run(table1_hbm)


def _prep_body(cnt_ref, x_ref, dinv_ref, xs0_ref, xs1_ref):
    ones = jnp.ones((NT, 1), jnp.float32)
    deg = lax.dot_general(cnt_ref[...], ones, (((0,), (0,)), ((), ())),
                          precision=lax.Precision.HIGHEST) + 1.0
    dinv = lax.rsqrt(deg)[:N]
    dinv_ref[...] = dinv
    xs = x_ref[...] * dinv
    xs0_ref[...] = xs[:, :DHALF]
    xs1_ref[...] = xs[:, DHALF:]


_prep = pl.pallas_call(
    _prep_body,
    out_shape=(
        jax.ShapeDtypeStruct((N, 1), jnp.float32),
        jax.ShapeDtypeStruct((N, DHALF), jnp.float32),
        jax.ShapeDtypeStruct((N, DHALF), jnp.float32),
    ),
)


def _mid_body(p_ref, xs0_ref, xs1_ref, dinv_ref, W1_ref, g1_ref, beta1_ref,
              W2_ref, zs0_ref, zs1_ref):
    dinv = dinv_ref[...]
    a0 = (p_ref[0] + xs0_ref[...]) * dinv
    a1 = (p_ref[1] + xs1_ref[...]) * dinv
    h = (jnp.dot(a0, W1_ref[:DHALF], preferred_element_type=jnp.float32)
         + jnp.dot(a1, W1_ref[DHALF:], preferred_element_type=jnp.float32))
    mu = jnp.mean(h, axis=0, keepdims=True)
    d = h - mu
    var = jnp.mean(d * d, axis=0, keepdims=True)
    hn = g1_ref[...] * (d * lax.rsqrt(var + EPS)) + beta1_ref[...]
    r = jnp.maximum(hn, 0.0)
    zs0_ref[...] = jnp.dot(r, W2_ref[:, :DHALF],
                           preferred_element_type=jnp.float32) * dinv
    zs1_ref[...] = jnp.dot(r, W2_ref[:, DHALF:],
                           preferred_element_type=jnp.float32) * dinv


_mid = pl.pallas_call(
    _mid_body,
    out_shape=(
        jax.ShapeDtypeStruct((N, DHALF), jnp.float32),
        jax.ShapeDtypeStruct((N, DHALF), jnp.float32),
    ),
)


def _fin_body(q_ref, zs0_ref, zs1_ref, dinv_ref, g2_ref, beta2_ref, out_ref):
    dinv = dinv_ref[...]
    for half, (q, zs) in enumerate(((q_ref[0], zs0_ref), (q_ref[1], zs1_ref))):
        o = (q + zs[...]) * dinv
        mu = jnp.mean(o, axis=0, keepdims=True)
        d = o - mu
        var = jnp.mean(d * d, axis=0, keepdims=True)
        sl = pl.ds(half * DHALF, DHALF)
        out_ref[:, sl] = (g2_ref[sl] * (d * lax.rsqrt(var + EPS))
                          + beta2_ref[sl])


_fin = pl.pallas_call(
    _fin_body,
    out_shape=jax.ShapeDtypeStruct((N, DOUT), jnp.float32),
)


def kernel(x, edge_index, W1, b1, g1, beta1, W2, b2, g2, beta2):
    del b1, b2
    src3d = edge_index[0].reshape(NS, NCHUNK, CH)
    dst3d = edge_index[1].reshape(NS, NCHUNK, CH)
    zpad = jnp.zeros((NPAD,), jnp.float32)
    z2d = jnp.zeros((N, DHALF), jnp.float32)
    cnts = _deg_kernel(edge_index[1], zpad)
    dinv, xs0, xs1 = _prep(cnts, x)
    p = _agg_kernel(xs0, xs1, src3d, dst3d, z2d)
    zs0, zs1 = _mid(p, xs0, xs1, dinv, W1, g1, beta1, W2)
    q = _agg_kernel(zs0, zs1, src3d, dst3d, z2d)
    return _fin(q, zs0, zs1, dinv, g2, beta2)
